# baseline (speedup 1.0000x reference)
import functools

import jax
import jax.numpy as jnp
from jax import lax
from jax.experimental import pallas as pl
from jax.experimental.pallas import tpu as pltpu
from jax.experimental.pallas import tpu_sc as plsc

V = 1000000
D = 64
B_TOK = 4096
S_TOK = 200
SCALE = 8.0
K1_W = 2048
K3_H = 512


def _dot_t(x, contract_dim):
    eye = jnp.eye(D, dtype=jnp.float32)
    return lax.dot_general(
        eye, x, (((1,), (contract_dim,)), ((), ())),
        precision=lax.Precision.HIGHEST,
        preferred_element_type=jnp.float32,
    )


def _k1_tc(t_t):
    def body(x_ref, o_ref):
        x = x_ref[...] * SCALE
        xt = lax.dot_general(
            x, jnp.eye(D, dtype=jnp.float32), (((0,), (0,)), ((), ())),
            precision=lax.Precision.HIGHEST,
            preferred_element_type=jnp.float32,
        )
        o_ref[...] = jnp.concatenate(
            [xt, jnp.zeros((K1_W, D), jnp.float32)], axis=1)

    grid = (V + K1_W - 1) // K1_W
    return pl.pallas_call(
        body,
        grid=(grid,),
        in_specs=[pl.BlockSpec((D, K1_W), lambda i: (0, i))],
        out_specs=pl.BlockSpec((K1_W, 128), lambda i: (i, 0)),
        out_shape=jax.ShapeDtypeStruct((V, 128), jnp.float32),
    )(t_t)


def _k3_tc(out2):
    nc = B_TOK // (2 * K3_H)

    def body(x_ref, o_ref):
        x = x_ref[...]
        o_ref[...] = jnp.concatenate(
            [_dot_t(x[:, :D], 1), _dot_t(x[:, D:], 1)], axis=1
        ).reshape(1, D, 2 * K3_H)

    return pl.pallas_call(
        body,
        grid=(S_TOK, nc),
        in_specs=[pl.BlockSpec((K3_H, 128), lambda s, c: (s * nc + c, 0))],
        out_specs=pl.BlockSpec((1, D, 2 * K3_H), lambda s, c: (s, 0, c)),
        out_shape=jax.ShapeDtypeStruct((S_TOK, D, B_TOK), jnp.float32),
    )(out2)


def _make_k2():
    info = plsc.get_sparse_core_info()
    nc, ns = info.num_cores, info.num_subcores
    mesh = plsc.VectorSubcoreMesh(core_axis_name="c", subcore_axis_name="s")
    half = B_TOK // 2

    @functools.partial(
        pl.kernel,
        mesh=mesh,
        out_type=jax.ShapeDtypeStruct((S_TOK * half, 128), jnp.float32),
        scratch_types=[
            pltpu.VMEM((S_TOK, 128), jnp.int32),
            pltpu.VMEM((128, 128), jnp.float32),
            pltpu.VMEM((128, 128), jnp.float32),
            pltpu.SemaphoreType.DMA,
            pltpu.SemaphoreType.DMA,
            pltpu.SemaphoreType.DMA,
            pltpu.SemaphoreType.DMA,
        ],
        compiler_params=pltpu.CompilerParams(use_tc_tiling_on_sc=False),
    )
    def k2(t2p, sidx, out2, idx_v, g0, g1, gs0, gs1, ws0, ws1):
        w = lax.axis_index("s") * nc + lax.axis_index("c")
        pltpu.sync_copy(sidx.at[:, w], idx_v)
        row0 = 512 * (w // 8) + 128 * (w % 4)
        col0 = D * ((w // 4) % 2)

        def fire_gather(s, grow, gsem):
            pltpu.async_copy(t2p.at[idx_v.at[s]], grow, gsem)

        def wait_gather(grow, gsem):
            pltpu.make_async_copy(t2p.at[idx_v.at[0]], grow, gsem).wait()

        def out_slice(s):
            return out2.at[pl.ds(s * half + row0, 128), pl.ds(col0, D)]

        def fire_write(s, grow, wsem):
            pltpu.async_copy(grow.at[:, pl.ds(0, D)], out_slice(s), wsem)

        def wait_write(grow, wsem):
            pltpu.make_async_copy(
                grow.at[:, pl.ds(0, D)], out_slice(0), wsem).wait()

        fire_gather(0, g0, gs0)

        @pl.loop(0, S_TOK, step=2)
        def _(a):
            @pl.when(a > 0)
            def _():
                wait_write(g1, ws1)

            fire_gather(a + 1, g1, gs1)
            wait_gather(g0, gs0)
            fire_write(a, g0, ws0)
            wait_write(g0, ws0)

            @pl.when(a + 2 < S_TOK)
            def _():
                fire_gather(a + 2, g0, gs0)

            wait_gather(g1, gs1)
            fire_write(a + 1, g1, ws1)

        wait_write(g1, ws1)

    return k2


def kernel(src, table):
    t_t = jnp.transpose(table)
    t2p = _k1_tc(t_t)
    sidx = jnp.transpose(src).astype(jnp.int32).reshape(S_TOK, 32, 128)
    out2 = _make_k2()(t2p, sidx)
    o3 = _k3_tc(out2)
    return jnp.transpose(o3, (2, 0, 1))


# speedup vs baseline: 1.2198x; 1.2198x over previous
import functools

import jax
import jax.numpy as jnp
from jax import lax
from jax.experimental import pallas as pl
from jax.experimental.pallas import tpu as pltpu
from jax.experimental.pallas import tpu_sc as plsc

V = 1000000
D = 64
B_TOK = 4096
S_TOK = 200
SCALE = 8.0
K1_W = 2048
K3_H = 512


def _dot_t(x, contract_dim):
    eye = jnp.eye(D, dtype=jnp.float32)
    return lax.dot_general(
        eye, x, (((1,), (contract_dim,)), ((), ())),
        precision=lax.Precision.DEFAULT,
        preferred_element_type=jnp.float32,
    )


def _k1_tc(t_t):
    def body(x_ref, o_ref):
        x = x_ref[...] * SCALE
        xt = lax.dot_general(
            x, jnp.eye(D, dtype=jnp.float32), (((0,), (0,)), ((), ())),
            precision=lax.Precision.DEFAULT,
            preferred_element_type=jnp.float32,
        )
        o_ref[...] = jnp.concatenate(
            [xt, jnp.zeros((K1_W, D), jnp.float32)], axis=1)

    grid = (V + K1_W - 1) // K1_W
    return pl.pallas_call(
        body,
        grid=(grid,),
        in_specs=[pl.BlockSpec((D, K1_W), lambda i: (0, i))],
        out_specs=pl.BlockSpec((K1_W, 128), lambda i: (i, 0)),
        out_shape=jax.ShapeDtypeStruct((V, 128), jnp.float32),
    )(t_t)


def _k3_tc(out2):
    nc = B_TOK // (2 * K3_H)

    def body(x_ref, o_ref):
        x = x_ref[...]
        o_ref[...] = jnp.concatenate(
            [_dot_t(x[:, :D], 1), _dot_t(x[:, D:], 1)], axis=1
        ).reshape(1, D, 2 * K3_H)

    return pl.pallas_call(
        body,
        grid=(S_TOK, nc),
        in_specs=[pl.BlockSpec((K3_H, 128), lambda s, c: (s * nc + c, 0))],
        out_specs=pl.BlockSpec((1, D, 2 * K3_H), lambda s, c: (s, 0, c)),
        out_shape=jax.ShapeDtypeStruct((S_TOK, D, B_TOK), jnp.float32),
    )(out2)


def _make_k2():
    info = plsc.get_sparse_core_info()
    nc, ns = info.num_cores, info.num_subcores
    mesh = plsc.VectorSubcoreMesh(core_axis_name="c", subcore_axis_name="s")
    half = B_TOK // 2

    @functools.partial(
        pl.kernel,
        mesh=mesh,
        out_type=jax.ShapeDtypeStruct((S_TOK * half, 128), jnp.float32),
        scratch_types=[
            pltpu.VMEM((S_TOK, 128), jnp.int32),
            pltpu.VMEM((128, 128), jnp.float32),
            pltpu.VMEM((128, 128), jnp.float32),
            pltpu.SemaphoreType.DMA,
            pltpu.SemaphoreType.DMA,
            pltpu.SemaphoreType.DMA,
            pltpu.SemaphoreType.DMA,
        ],
        compiler_params=pltpu.CompilerParams(use_tc_tiling_on_sc=False),
    )
    def k2(t2p, sidx, out2, idx_v, g0, g1, gs0, gs1, ws0, ws1):
        w = lax.axis_index("s") * nc + lax.axis_index("c")
        pltpu.sync_copy(sidx.at[:, w], idx_v)
        row0 = 512 * (w // 8) + 128 * (w % 4)
        col0 = D * ((w // 4) % 2)

        def fire_gather(s, grow, gsem):
            pltpu.async_copy(t2p.at[idx_v.at[s]], grow, gsem)

        def wait_gather(grow, gsem):
            pltpu.make_async_copy(t2p.at[idx_v.at[0]], grow, gsem).wait()

        def out_slice(s):
            return out2.at[pl.ds(s * half + row0, 128), pl.ds(col0, D)]

        def fire_write(s, grow, wsem):
            pltpu.async_copy(grow.at[:, pl.ds(0, D)], out_slice(s), wsem)

        def wait_write(grow, wsem):
            pltpu.make_async_copy(
                grow.at[:, pl.ds(0, D)], out_slice(0), wsem).wait()

        fire_gather(0, g0, gs0)

        @pl.loop(0, S_TOK, step=2)
        def _(a):
            @pl.when(a > 0)
            def _():
                wait_write(g1, ws1)

            fire_gather(a + 1, g1, gs1)
            wait_gather(g0, gs0)
            fire_write(a, g0, ws0)
            wait_write(g0, ws0)

            @pl.when(a + 2 < S_TOK)
            def _():
                fire_gather(a + 2, g0, gs0)

            wait_gather(g1, gs1)
            fire_write(a + 1, g1, ws1)

        wait_write(g1, ws1)

    return k2


def kernel(src, table):
    t_t = jnp.transpose(table)
    t2p = _k1_tc(t_t)
    sidx = jnp.transpose(src).astype(jnp.int32).reshape(S_TOK, 32, 128)
    out2 = _make_k2()(t2p, sidx)
    o3 = _k3_tc(out2)
    return jnp.transpose(o3, (2, 0, 1))


# speedup vs baseline: 1.2761x; 1.0462x over previous
import functools
import math

import jax
import jax.numpy as jnp
from jax import lax
from jax.experimental import pallas as pl
from jax.experimental.pallas import tpu as pltpu
from jax.experimental.pallas import tpu_sc as plsc

D_MODEL = 64
LANES = 16
SUB = 128
CHUNK_SUBS = 5
CHUNK = SUB * CHUNK_SUBS


def _make_gather(num_idx: int, scale: float):
    info = plsc.get_sparse_core_info()
    nc, ns = info.num_cores, info.num_subcores
    nw = nc * ns
    bpw = num_idx // nw
    assert num_idx % (nw * 2 * CHUNK) == 0
    nsub = bpw // SUB
    nchunks = bpw // CHUNK

    mesh = plsc.VectorSubcoreMesh(core_axis_name="c", subcore_axis_name="s")

    @functools.partial(
        pl.kernel,
        mesh=mesh,
        out_type=jax.ShapeDtypeStruct((num_idx, D_MODEL), jnp.float32),
        scratch_types=[
            pltpu.VMEM((nsub, SUB), jnp.int32),
            pltpu.VMEM((CHUNK, D_MODEL), jnp.float32),
            pltpu.VMEM((CHUNK, D_MODEL), jnp.float32),
            pltpu.SemaphoreType.DMA,
            pltpu.SemaphoreType.DMA,
            pltpu.SemaphoreType.DMA,
            pltpu.SemaphoreType.DMA,
        ],
        compiler_params=pltpu.CompilerParams(use_tc_tiling_on_sc=False),
    )
    def gather_kernel(
        table_hbm, idx_hbm, out_hbm, idx_v, rows0, rows1, gsem0, gsem1, wsem0, wsem1
    ):
        wid = lax.axis_index("s") * nc + lax.axis_index("c")
        pltpu.sync_copy(idx_hbm.at[wid], idx_v)

        def fire_gathers(g, rows, gsem):
            for j in range(CHUNK_SUBS):
                pltpu.async_copy(
                    table_hbm.at[idx_v.at[g * CHUNK_SUBS + j]],
                    rows.at[pl.ds(j * SUB, SUB)],
                    gsem,
                )

        def wait_gathers(rows, gsem):
            for j in range(CHUNK_SUBS):
                pltpu.make_async_copy(
                    table_hbm.at[idx_v.at[j]],
                    rows.at[pl.ds(j * SUB, SUB)],
                    gsem,
                ).wait()

        def scale_rows(rows):
            @plsc.parallel_loop(0, CHUNK, unroll=4)
            def _(r):
                for t in range(D_MODEL // LANES):
                    sl = pl.ds(t * LANES, LANES)
                    rows[r, sl] = rows[r, sl] * scale

        def out_slice(g):
            return out_hbm.at[pl.ds(wid * bpw + g * CHUNK, CHUNK)]

        def fire_write(g, rows, wsem):
            pltpu.async_copy(rows, out_slice(g), wsem)

        def wait_write(rows, wsem):
            pltpu.make_async_copy(rows, out_slice(0), wsem).wait()

        fire_gathers(0, rows0, gsem0)

        @pl.loop(0, nchunks, step=2)
        def _(a):
            @pl.when(a > 0)
            def _():
                wait_write(rows1, wsem1)

            fire_gathers(a + 1, rows1, gsem1)
            wait_gathers(rows0, gsem0)
            scale_rows(rows0)
            fire_write(a, rows0, wsem0)
            wait_write(rows0, wsem0)

            @pl.when(a + 2 < nchunks)
            def _():
                fire_gathers(a + 2, rows0, gsem0)

            wait_gathers(rows1, gsem1)
            scale_rows(rows1)
            fire_write(a + 1, rows1, wsem1)

        wait_write(rows1, wsem1)

    return gather_kernel


def kernel(src, table):
    b, s = src.shape
    num_idx = b * s
    info = plsc.get_sparse_core_info()
    nw = info.num_cores * info.num_subcores
    idx = src.reshape(nw, (num_idx // nw) // SUB, SUB).astype(jnp.int32)
    scale = math.sqrt(table.shape[1])
    out = _make_gather(num_idx, scale)(table, idx)
    return out.reshape(b, s, table.shape[1])


# speedup vs baseline: 1.8314x; 1.4352x over previous
import functools

import jax
import jax.numpy as jnp
from jax import lax
from jax.experimental import pallas as pl
from jax.experimental.pallas import tpu as pltpu
from jax.experimental.pallas import tpu_sc as plsc

V = 1000000
D = 64
B_TOK = 4096
S_TOK = 200
SCALE = 8.0
K1_W = 4096
K3_H = 2048


def _k1_tc(t_t):
    def body(x_ref, o_ref):
        x = x_ref[...] * SCALE
        o_ref[...] = jnp.concatenate(
            [jnp.transpose(x), jnp.zeros((K1_W, D), jnp.float32)], axis=1)

    grid = (V + K1_W - 1) // K1_W
    return pl.pallas_call(
        body,
        grid=(grid,),
        in_specs=[pl.BlockSpec((D, K1_W), lambda i: (0, i))],
        out_specs=pl.BlockSpec((K1_W, 128), lambda i: (i, 0)),
        out_shape=jax.ShapeDtypeStruct((V, 128), jnp.float32),
    )(t_t)


def _k3_tc(out2):
    def body(x_ref, o_ref):
        x = x_ref[...]
        o_ref[...] = jnp.concatenate(
            [jnp.transpose(x[:, :D]), jnp.transpose(x[:, D:])], axis=1
        ).reshape(1, D, B_TOK)

    return pl.pallas_call(
        body,
        grid=(S_TOK,),
        in_specs=[pl.BlockSpec((K3_H, 128), lambda s: (s, 0))],
        out_specs=pl.BlockSpec((1, D, B_TOK), lambda s: (s, 0, 0)),
        out_shape=jax.ShapeDtypeStruct((S_TOK, D, B_TOK), jnp.float32),
    )(out2)


def _make_k2():
    info = plsc.get_sparse_core_info()
    nc, ns = info.num_cores, info.num_subcores
    mesh = plsc.VectorSubcoreMesh(core_axis_name="c", subcore_axis_name="s")
    half = B_TOK // 2

    @functools.partial(
        pl.kernel,
        mesh=mesh,
        out_type=jax.ShapeDtypeStruct((S_TOK * half, 128), jnp.float32),
        scratch_types=[
            pltpu.VMEM((S_TOK, 128), jnp.int32),
            pltpu.VMEM((128, 128), jnp.float32),
            pltpu.VMEM((128, 128), jnp.float32),
            pltpu.SemaphoreType.DMA,
            pltpu.SemaphoreType.DMA,
            pltpu.SemaphoreType.DMA,
            pltpu.SemaphoreType.DMA,
        ],
        compiler_params=pltpu.CompilerParams(use_tc_tiling_on_sc=False),
    )
    def k2(t2p, sidx, out2, idx_v, g0, g1, gs0, gs1, ws0, ws1):
        w = lax.axis_index("s") * nc + lax.axis_index("c")
        pltpu.sync_copy(sidx.at[:, w], idx_v)
        row0 = 128 * (w % 16)
        col0 = D * (w // 16)

        def fire_gather(s, grow, gsem):
            pltpu.async_copy(t2p.at[idx_v.at[s]], grow, gsem)

        def wait_gather(grow, gsem):
            pltpu.make_async_copy(t2p.at[idx_v.at[0]], grow, gsem).wait()

        def out_slice(s):
            return out2.at[pl.ds(s * half + row0, 128), pl.ds(col0, D)]

        def fire_write(s, grow, wsem):
            pltpu.async_copy(grow.at[:, pl.ds(0, D)], out_slice(s), wsem)

        def wait_write(grow, wsem):
            pltpu.make_async_copy(
                grow.at[:, pl.ds(0, D)], out_slice(0), wsem).wait()

        fire_gather(0, g0, gs0)

        @pl.loop(0, S_TOK, step=2)
        def _(a):
            @pl.when(a > 0)
            def _():
                wait_write(g1, ws1)

            fire_gather(a + 1, g1, gs1)
            wait_gather(g0, gs0)
            fire_write(a, g0, ws0)
            wait_write(g0, ws0)

            @pl.when(a + 2 < S_TOK)
            def _():
                fire_gather(a + 2, g0, gs0)

            wait_gather(g1, gs1)
            fire_write(a + 1, g1, ws1)

        wait_write(g1, ws1)

    return k2


def kernel(src, table):
    t_t = jnp.transpose(table)
    t2p = _k1_tc(t_t)
    sidx = jnp.transpose(src).astype(jnp.int32).reshape(S_TOK, 32, 128)
    out2 = _make_k2()(t2p, sidx)
    o3 = _k3_tc(out2)
    return jnp.transpose(o3, (2, 0, 1))


# speedup vs baseline: 1.8363x; 1.0027x over previous
import functools

import jax
import jax.numpy as jnp
from jax import lax
from jax.experimental import pallas as pl
from jax.experimental.pallas import tpu as pltpu
from jax.experimental.pallas import tpu_sc as plsc

V = 1000000
D = 64
B_TOK = 4096
S_TOK = 200
SCALE = 8.0
K1_W = 4096
K3_S = 1


def _k1_tc(t_t):
    def body(x_ref, o_ref):
        x = x_ref[...] * SCALE
        o_ref[...] = jnp.concatenate(
            [jnp.transpose(x), jnp.zeros((K1_W, D), jnp.float32)], axis=1)

    grid = (V + K1_W - 1) // K1_W
    return pl.pallas_call(
        body,
        grid=(grid,),
        in_specs=[pl.BlockSpec((D, K1_W), lambda i: (0, i))],
        out_specs=pl.BlockSpec((K1_W, 128), lambda i: (i, 0)),
        out_shape=jax.ShapeDtypeStruct((V, 128), jnp.float32),
    )(t_t)


def _k3_tc(out2):
    h = K3_S * B_TOK // 2

    def body(x_ref, o_ref):
        x = x_ref[...]
        for j in range(K3_S):
            xs = x[j * (B_TOK // 2):(j + 1) * (B_TOK // 2), :]
            o_ref[j, :, :] = jnp.concatenate(
                [jnp.transpose(xs[:, :D]), jnp.transpose(xs[:, D:])], axis=1)

    return pl.pallas_call(
        body,
        grid=(S_TOK // K3_S,),
        in_specs=[pl.BlockSpec((h, 128), lambda s: (s, 0))],
        out_specs=pl.BlockSpec((K3_S, D, B_TOK), lambda s: (s, 0, 0)),
        out_shape=jax.ShapeDtypeStruct((S_TOK, D, B_TOK), jnp.float32),
    )(out2)


def _make_k2():
    info = plsc.get_sparse_core_info()
    nc, ns = info.num_cores, info.num_subcores
    mesh = plsc.VectorSubcoreMesh(core_axis_name="c", subcore_axis_name="s")
    half = B_TOK // 2

    @functools.partial(
        pl.kernel,
        mesh=mesh,
        out_type=jax.ShapeDtypeStruct((S_TOK * half, 128), jnp.float32),
        scratch_types=[
            pltpu.VMEM((S_TOK, 128), jnp.int32),
            pltpu.VMEM((128, 128), jnp.float32),
            pltpu.VMEM((128, 128), jnp.float32),
            pltpu.SemaphoreType.DMA,
            pltpu.SemaphoreType.DMA,
            pltpu.SemaphoreType.DMA,
            pltpu.SemaphoreType.DMA,
        ],
        compiler_params=pltpu.CompilerParams(use_tc_tiling_on_sc=False),
    )
    def k2(t2p, sidx, out2, idx_v, g0, g1, gs0, gs1, ws0, ws1):
        w = lax.axis_index("s") * nc + lax.axis_index("c")
        pltpu.sync_copy(sidx.at[:, w], idx_v)
        row0 = 128 * (w % 16)
        col0 = D * (w // 16)

        def fire_gather(s, grow, gsem):
            pltpu.async_copy(t2p.at[idx_v.at[s]], grow, gsem)

        def wait_gather(grow, gsem):
            pltpu.make_async_copy(t2p.at[idx_v.at[0]], grow, gsem).wait()

        def out_slice(s):
            return out2.at[pl.ds(s * half + row0, 128), pl.ds(col0, D)]

        def fire_write(s, grow, wsem):
            pltpu.async_copy(grow.at[:, pl.ds(0, D)], out_slice(s), wsem)

        def wait_write(grow, wsem):
            pltpu.make_async_copy(
                grow.at[:, pl.ds(0, D)], out_slice(0), wsem).wait()

        fire_gather(0, g0, gs0)

        @pl.loop(0, S_TOK, step=2)
        def _(a):
            @pl.when(a > 0)
            def _():
                wait_write(g1, ws1)

            fire_gather(a + 1, g1, gs1)
            wait_gather(g0, gs0)
            fire_write(a, g0, ws0)
            wait_write(g0, ws0)

            @pl.when(a + 2 < S_TOK)
            def _():
                fire_gather(a + 2, g0, gs0)

            wait_gather(g1, gs1)
            fire_write(a + 1, g1, ws1)

        wait_write(g1, ws1)

    return k2


def kernel(src, table):
    t_t = jnp.transpose(table)
    t2p = _k1_tc(t_t)
    sidx = jnp.transpose(src).astype(jnp.int32).reshape(S_TOK, 32, 128)
    out2 = _make_k2()(t2p, sidx)
    o3 = _k3_tc(out2)
    return jnp.transpose(o3, (2, 0, 1))


# speedup vs baseline: 2.1530x; 1.1724x over previous
import functools

import jax
import jax.numpy as jnp
from jax import lax
from jax.experimental import pallas as pl
from jax.experimental.pallas import tpu as pltpu
from jax.experimental.pallas import tpu_sc as plsc

V = 1000000
D = 64
B_TOK = 4096
S_TOK = 200
SCALE = 8.0
K1_W = 8192
K3_S = 2


def _k1_tc(t_t):
    def body(x_ref, o_ref):
        x = x_ref[...] * SCALE
        o_ref[...] = jnp.concatenate(
            [jnp.transpose(x), jnp.zeros((K1_W, D), jnp.float32)], axis=1)

    grid = (V + K1_W - 1) // K1_W
    return pl.pallas_call(
        body,
        grid=(grid,),
        in_specs=[pl.BlockSpec((D, K1_W), lambda i: (0, i))],
        out_specs=pl.BlockSpec((K1_W, 128), lambda i: (i, 0)),
        out_shape=jax.ShapeDtypeStruct((V, 128), jnp.float32),
    )(t_t)


def _k3_tc(out2):
    h = K3_S * B_TOK // 2

    def body(x_ref, o_ref):
        x = x_ref[...]
        for j in range(K3_S):
            xs = x[j * (B_TOK // 2):(j + 1) * (B_TOK // 2), :]
            o_ref[j, :, :] = jnp.concatenate(
                [jnp.transpose(xs[:, :D]), jnp.transpose(xs[:, D:])], axis=1)

    return pl.pallas_call(
        body,
        grid=(S_TOK // K3_S,),
        in_specs=[pl.BlockSpec((h, 128), lambda s: (s, 0))],
        out_specs=pl.BlockSpec((K3_S, D, B_TOK), lambda s: (s, 0, 0)),
        out_shape=jax.ShapeDtypeStruct((S_TOK, D, B_TOK), jnp.float32),
    )(out2)


def _make_k2():
    info = plsc.get_sparse_core_info()
    nc, ns = info.num_cores, info.num_subcores
    mesh = plsc.VectorSubcoreMesh(core_axis_name="c", subcore_axis_name="s")
    half = B_TOK // 2

    @functools.partial(
        pl.kernel,
        mesh=mesh,
        out_type=jax.ShapeDtypeStruct((S_TOK * half, 128), jnp.float32),
        scratch_types=[
            pltpu.VMEM((S_TOK, 128), jnp.int32),
            pltpu.VMEM((128, 128), jnp.float32),
            pltpu.VMEM((128, 128), jnp.float32),
            pltpu.SemaphoreType.DMA,
            pltpu.SemaphoreType.DMA,
            pltpu.SemaphoreType.DMA,
            pltpu.SemaphoreType.DMA,
        ],
        compiler_params=pltpu.CompilerParams(use_tc_tiling_on_sc=False),
    )
    def k2(t2p, sidx, out2, idx_v, g0, g1, gs0, gs1, ws0, ws1):
        w = lax.axis_index("s") * nc + lax.axis_index("c")
        pltpu.sync_copy(sidx.at[:, w], idx_v)
        row0 = 128 * (w % 16)
        col0 = D * (w // 16)

        def fire_gather(s, grow, gsem):
            pltpu.async_copy(t2p.at[idx_v.at[s]], grow, gsem)

        def wait_gather(grow, gsem):
            pltpu.make_async_copy(t2p.at[idx_v.at[0]], grow, gsem).wait()

        def out_slice(s):
            return out2.at[pl.ds(s * half + row0, 128), pl.ds(col0, D)]

        def fire_write(s, grow, wsem):
            pltpu.async_copy(grow.at[:, pl.ds(0, D)], out_slice(s), wsem)

        def wait_write(grow, wsem):
            pltpu.make_async_copy(
                grow.at[:, pl.ds(0, D)], out_slice(0), wsem).wait()

        fire_gather(0, g0, gs0)

        @pl.loop(0, S_TOK, step=2)
        def _(a):
            @pl.when(a > 0)
            def _():
                wait_write(g1, ws1)

            fire_gather(a + 1, g1, gs1)
            wait_gather(g0, gs0)
            fire_write(a, g0, ws0)
            wait_write(g0, ws0)

            @pl.when(a + 2 < S_TOK)
            def _():
                fire_gather(a + 2, g0, gs0)

            wait_gather(g1, gs1)
            fire_write(a + 1, g1, ws1)

        wait_write(g1, ws1)

    return k2


def kernel(src, table):
    t_t = jnp.transpose(table)
    t2p = _k1_tc(t_t)
    sidx = jnp.transpose(src).astype(jnp.int32).reshape(S_TOK, 32, 128)
    out2 = _make_k2()(t2p, sidx)
    o3 = _k3_tc(out2)
    return jnp.transpose(o3, (2, 0, 1))


# speedup vs baseline: 2.3005x; 1.0685x over previous
import functools

import jax
import jax.numpy as jnp
from jax import lax
from jax.experimental import pallas as pl
from jax.experimental.pallas import tpu as pltpu
from jax.experimental.pallas import tpu_sc as plsc

V = 1000000
D = 64
B_TOK = 4096
S_TOK = 200
SCALE = 8.0
K1_W = 16384
K3_S = 4


def _k1_tc(t_t):
    def body(x_ref, o_ref):
        x = x_ref[...] * SCALE
        o_ref[...] = jnp.concatenate(
            [jnp.transpose(x), jnp.zeros((K1_W, D), jnp.float32)], axis=1)

    grid = (V + K1_W - 1) // K1_W
    return pl.pallas_call(
        body,
        grid=(grid,),
        in_specs=[pl.BlockSpec((D, K1_W), lambda i: (0, i))],
        out_specs=pl.BlockSpec((K1_W, 128), lambda i: (i, 0)),
        out_shape=jax.ShapeDtypeStruct((V, 128), jnp.float32),
    )(t_t)


def _k3_tc(out2):
    h = K3_S * B_TOK // 2

    def body(x_ref, o_ref):
        x = x_ref[...]
        for j in range(K3_S):
            xs = x[j * (B_TOK // 2):(j + 1) * (B_TOK // 2), :]
            o_ref[j, :, :] = jnp.concatenate(
                [jnp.transpose(xs[:, :D]), jnp.transpose(xs[:, D:])], axis=1)

    return pl.pallas_call(
        body,
        grid=(S_TOK // K3_S,),
        in_specs=[pl.BlockSpec((h, 128), lambda s: (s, 0))],
        out_specs=pl.BlockSpec((K3_S, D, B_TOK), lambda s: (s, 0, 0)),
        out_shape=jax.ShapeDtypeStruct((S_TOK, D, B_TOK), jnp.float32),
    )(out2)


def _make_k2():
    info = plsc.get_sparse_core_info()
    nc, ns = info.num_cores, info.num_subcores
    mesh = plsc.VectorSubcoreMesh(core_axis_name="c", subcore_axis_name="s")
    half = B_TOK // 2

    @functools.partial(
        pl.kernel,
        mesh=mesh,
        out_type=jax.ShapeDtypeStruct((S_TOK * half, 128), jnp.float32),
        scratch_types=[
            pltpu.VMEM((S_TOK, 128), jnp.int32),
            pltpu.VMEM((128, 128), jnp.float32),
            pltpu.VMEM((128, 128), jnp.float32),
            pltpu.SemaphoreType.DMA,
            pltpu.SemaphoreType.DMA,
            pltpu.SemaphoreType.DMA,
            pltpu.SemaphoreType.DMA,
        ],
        compiler_params=pltpu.CompilerParams(use_tc_tiling_on_sc=False),
    )
    def k2(t2p, sidx, out2, idx_v, g0, g1, gs0, gs1, ws0, ws1):
        w = lax.axis_index("s") * nc + lax.axis_index("c")
        pltpu.sync_copy(sidx.at[:, w], idx_v)
        row0 = 128 * (w % 16)
        col0 = D * (w // 16)

        def fire_gather(s, grow, gsem):
            pltpu.async_copy(t2p.at[idx_v.at[s]], grow, gsem)

        def wait_gather(grow, gsem):
            pltpu.make_async_copy(t2p.at[idx_v.at[0]], grow, gsem).wait()

        def out_slice(s):
            return out2.at[pl.ds(s * half + row0, 128), pl.ds(col0, D)]

        def fire_write(s, grow, wsem):
            pltpu.async_copy(grow.at[:, pl.ds(0, D)], out_slice(s), wsem)

        def wait_write(grow, wsem):
            pltpu.make_async_copy(
                grow.at[:, pl.ds(0, D)], out_slice(0), wsem).wait()

        fire_gather(0, g0, gs0)

        @pl.loop(0, S_TOK, step=2)
        def _(a):
            @pl.when(a > 0)
            def _():
                wait_write(g1, ws1)

            fire_gather(a + 1, g1, gs1)
            wait_gather(g0, gs0)
            fire_write(a, g0, ws0)
            wait_write(g0, ws0)

            @pl.when(a + 2 < S_TOK)
            def _():
                fire_gather(a + 2, g0, gs0)

            wait_gather(g1, gs1)
            fire_write(a + 1, g1, ws1)

        wait_write(g1, ws1)

    return k2


def kernel(src, table):
    t_t = jnp.transpose(table)
    t2p = _k1_tc(t_t)
    sidx = jnp.transpose(src).astype(jnp.int32).reshape(S_TOK, 32, 128)
    out2 = _make_k2()(t2p, sidx)
    o3 = _k3_tc(out2)
    return jnp.transpose(o3, (2, 0, 1))


# speedup vs baseline: 2.3620x; 1.0267x over previous
import functools

import jax
import jax.numpy as jnp
from jax import lax
from jax.experimental import pallas as pl
from jax.experimental.pallas import tpu as pltpu
from jax.experimental.pallas import tpu_sc as plsc

V = 1000000
D = 64
B_TOK = 4096
S_TOK = 200
SCALE = 8.0
K1_W = 32768
K3_S = 8


def _k1_tc(t_t):
    def body(x_ref, o_ref):
        x = x_ref[...] * SCALE
        o_ref[...] = jnp.concatenate(
            [jnp.transpose(x), jnp.zeros((K1_W, D), jnp.float32)], axis=1)

    grid = (V + K1_W - 1) // K1_W
    return pl.pallas_call(
        body,
        grid=(grid,),
        in_specs=[pl.BlockSpec((D, K1_W), lambda i: (0, i))],
        out_specs=pl.BlockSpec((K1_W, 128), lambda i: (i, 0)),
        out_shape=jax.ShapeDtypeStruct((V, 128), jnp.float32),
    )(t_t)


def _k3_tc(out2):
    h = K3_S * B_TOK // 2

    def body(x_ref, o_ref):
        x = x_ref[...]
        for j in range(K3_S):
            xs = x[j * (B_TOK // 2):(j + 1) * (B_TOK // 2), :]
            o_ref[j, :, :] = jnp.concatenate(
                [jnp.transpose(xs[:, :D]), jnp.transpose(xs[:, D:])], axis=1)

    return pl.pallas_call(
        body,
        grid=(S_TOK // K3_S,),
        in_specs=[pl.BlockSpec((h, 128), lambda s: (s, 0))],
        out_specs=pl.BlockSpec((K3_S, D, B_TOK), lambda s: (s, 0, 0)),
        out_shape=jax.ShapeDtypeStruct((S_TOK, D, B_TOK), jnp.float32),
    )(out2)


def _make_k2():
    info = plsc.get_sparse_core_info()
    nc, ns = info.num_cores, info.num_subcores
    mesh = plsc.VectorSubcoreMesh(core_axis_name="c", subcore_axis_name="s")
    half = B_TOK // 2

    @functools.partial(
        pl.kernel,
        mesh=mesh,
        out_type=jax.ShapeDtypeStruct((S_TOK * half, 128), jnp.float32),
        scratch_types=[
            pltpu.VMEM((S_TOK, 128), jnp.int32),
            pltpu.VMEM((128, 128), jnp.float32),
            pltpu.VMEM((128, 128), jnp.float32),
            pltpu.SemaphoreType.DMA,
            pltpu.SemaphoreType.DMA,
            pltpu.SemaphoreType.DMA,
            pltpu.SemaphoreType.DMA,
        ],
        compiler_params=pltpu.CompilerParams(use_tc_tiling_on_sc=False),
    )
    def k2(t2p, sidx, out2, idx_v, g0, g1, gs0, gs1, ws0, ws1):
        w = lax.axis_index("s") * nc + lax.axis_index("c")
        pltpu.sync_copy(sidx.at[:, w], idx_v)
        row0 = 128 * (w % 16)
        col0 = D * (w // 16)

        def fire_gather(s, grow, gsem):
            pltpu.async_copy(t2p.at[idx_v.at[s]], grow, gsem)

        def wait_gather(grow, gsem):
            pltpu.make_async_copy(t2p.at[idx_v.at[0]], grow, gsem).wait()

        def out_slice(s):
            return out2.at[pl.ds(s * half + row0, 128), pl.ds(col0, D)]

        def fire_write(s, grow, wsem):
            pltpu.async_copy(grow.at[:, pl.ds(0, D)], out_slice(s), wsem)

        def wait_write(grow, wsem):
            pltpu.make_async_copy(
                grow.at[:, pl.ds(0, D)], out_slice(0), wsem).wait()

        fire_gather(0, g0, gs0)

        @pl.loop(0, S_TOK, step=2)
        def _(a):
            @pl.when(a > 0)
            def _():
                wait_write(g1, ws1)

            fire_gather(a + 1, g1, gs1)
            wait_gather(g0, gs0)
            fire_write(a, g0, ws0)
            wait_write(g0, ws0)

            @pl.when(a + 2 < S_TOK)
            def _():
                fire_gather(a + 2, g0, gs0)

            wait_gather(g1, gs1)
            fire_write(a + 1, g1, ws1)

        wait_write(g1, ws1)

    return k2


def kernel(src, table):
    t_t = jnp.transpose(table)
    t2p = _k1_tc(t_t)
    sidx = jnp.transpose(src).astype(jnp.int32).reshape(S_TOK, 32, 128)
    out2 = _make_k2()(t2p, sidx)
    o3 = _k3_tc(out2)
    return jnp.transpose(o3, (2, 0, 1))


# speedup vs baseline: 2.4618x; 1.0423x over previous
import functools

import jax
import jax.numpy as jnp
from jax import lax
from jax.experimental import pallas as pl
from jax.experimental.pallas import tpu as pltpu
from jax.experimental.pallas import tpu_sc as plsc

V = 1000000
D = 64
B_TOK = 4096
S_TOK = 200
SCALE = 8.0
K1_W = 32768
K3_S = 8


def _k1_tc(t_t):
    def body(x_ref, o_ref):
        x = x_ref[...] * SCALE
        o_ref[...] = jnp.concatenate(
            [jnp.transpose(x), jnp.zeros((K1_W, D), jnp.float32)], axis=1)

    grid = (V + K1_W - 1) // K1_W
    return pl.pallas_call(
        body,
        grid=(grid,),
        in_specs=[pl.BlockSpec((D, K1_W), lambda i: (0, i))],
        out_specs=pl.BlockSpec((K1_W, 128), lambda i: (i, 0)),
        out_shape=jax.ShapeDtypeStruct((V, 128), jnp.float32),
    )(t_t)


def _k3_tc(out2):
    h = K3_S * B_TOK // 2

    def body(x_ref, o_ref):
        x = x_ref[...]
        for j in range(K3_S):
            xs = x[j * (B_TOK // 2):(j + 1) * (B_TOK // 2), :]
            o_ref[j, :, :] = jnp.concatenate(
                [jnp.transpose(xs[:, :D]), jnp.transpose(xs[:, D:])], axis=1)

    return pl.pallas_call(
        body,
        grid=(S_TOK // K3_S,),
        in_specs=[pl.BlockSpec((h, 128), lambda s: (s, 0))],
        out_specs=pl.BlockSpec((K3_S, D, B_TOK), lambda s: (s, 0, 0)),
        out_shape=jax.ShapeDtypeStruct((S_TOK, D, B_TOK), jnp.float32),
    )(out2)


def _make_k2():
    info = plsc.get_sparse_core_info()
    nc, ns = info.num_cores, info.num_subcores
    mesh = plsc.VectorSubcoreMesh(core_axis_name="c", subcore_axis_name="s")
    half = B_TOK // 2

    @functools.partial(
        pl.kernel,
        mesh=mesh,
        out_type=jax.ShapeDtypeStruct((S_TOK * half, 128), jnp.float32),
        scratch_types=[
            pltpu.VMEM((S_TOK, 128), jnp.int32),
            pltpu.VMEM((128, 128), jnp.float32),
            pltpu.VMEM((128, 128), jnp.float32),
            pltpu.VMEM((128, 128), jnp.float32),
            pltpu.VMEM((128, 128), jnp.float32),
            pltpu.SemaphoreType.DMA,
            pltpu.SemaphoreType.DMA,
            pltpu.SemaphoreType.DMA,
            pltpu.SemaphoreType.DMA,
            pltpu.SemaphoreType.DMA,
            pltpu.SemaphoreType.DMA,
            pltpu.SemaphoreType.DMA,
            pltpu.SemaphoreType.DMA,
        ],
        compiler_params=pltpu.CompilerParams(use_tc_tiling_on_sc=False),
    )
    def k2(t2p, sidx, out2, idx_v,
           g0, g1, g2, g3, gs0, gs1, gs2, gs3, ws0, ws1, ws2, ws3):
        w = lax.axis_index("s") * nc + lax.axis_index("c")
        pltpu.sync_copy(sidx.at[:, w], idx_v)
        row0 = 128 * (w % 16)
        col0 = D * (w // 16)
        bufs = (g0, g1, g2, g3)
        gsems = (gs0, gs1, gs2, gs3)
        wsems = (ws0, ws1, ws2, ws3)

        def fire_gather(s, j):
            pltpu.async_copy(t2p.at[idx_v.at[s]], bufs[j], gsems[j])

        def wait_gather(j):
            pltpu.make_async_copy(
                t2p.at[idx_v.at[0]], bufs[j], gsems[j]).wait()

        def out_slice(s):
            return out2.at[pl.ds(s * half + row0, 128), pl.ds(col0, D)]

        def fire_write(s, j):
            pltpu.async_copy(bufs[j].at[:, pl.ds(0, D)], out_slice(s), wsems[j])

        def wait_write(j):
            pltpu.make_async_copy(
                bufs[j].at[:, pl.ds(0, D)], out_slice(0), wsems[j]).wait()

        for j in range(3):
            fire_gather(j, j)

        @pl.loop(0, S_TOK, step=4)
        def _(a):
            for j in range(4):
                s = a + j
                jn = (j + 3) % 4
                wait_gather(j)
                fire_write(s, j)
                if j == 0:
                    @pl.when(a > 0)
                    def _():
                        wait_write(jn)
                else:
                    wait_write(jn)

                @pl.when(s + 3 < S_TOK)
                def _():
                    fire_gather(s + 3, jn)

        wait_write(3)

    return k2


def kernel(src, table):
    t_t = jnp.transpose(table)
    t2p = _k1_tc(t_t)
    sidx = jnp.transpose(src).astype(jnp.int32).reshape(S_TOK, 32, 128)
    out2 = _make_k2()(t2p, sidx)
    o3 = _k3_tc(out2)
    return jnp.transpose(o3, (2, 0, 1))
